# baseline (speedup 1.0000x reference)
import jax
import jax.numpy as jnp
from jax import lax
from jax.experimental import pallas as pl
from jax.experimental.pallas import tpu as pltpu
from jax.experimental.pallas import tpu_sc as plsc

NUM_CLASSES = 1000000
EMBED = 32
HIST = 50
BATCH = 16384

NC = 2
NS = 16
NW = NC * NS

B_PER_W = BATCH // NW
IDX_SUB = 128
N_SUB = B_PER_W // IDX_SUB
OB_PITCH = B_PER_W + 1
NBUF = 2
N_ROUNDS = -(-HIST // NBUF)

TC_COLS = 32768
TC_SUB = TC_COLS // 4
TC_GRID = -(-NUM_CLASSES // TC_COLS)
N_PAD = TC_GRID * TC_COLS


def _tc_compact_kernel(w_ref, o_ref):
    for j in range(4):
        o_ref[:, 32 * j:32 * (j + 1)] = w_ref[:, TC_SUB * j:TC_SUB * (j + 1)].T


def _compact_table(Wt):
    return pl.pallas_call(
        _tc_compact_kernel,
        grid=(TC_GRID,),
        in_specs=[pl.BlockSpec((EMBED, TC_COLS), lambda i: (0, i))],
        out_specs=pl.BlockSpec((TC_SUB, 4 * EMBED), lambda i: (i, 0)),
        out_shape=jax.ShapeDtypeStruct(
            (N_PAD // 4, 4 * EMBED), jnp.float32),
    )(Wt)


def _sc_kernel(x_hbm, w_hbm, out_hbm, idx_v, *rest):
    stagings = rest[0:NBUF]
    outbufs = rest[NBUF:2 * NBUF]
    sem_g = rest[2 * NBUF:3 * NBUF]
    sem_o = rest[3 * NBUF:4 * NBUF]

    wid = lax.axis_index("s") * NC + lax.axis_index("c")

    pltpu.sync_copy(x_hbm.at[:, wid], idx_v)

    def rbody(l, carry):
        for k in range(N_SUB):
            for g in range(IDX_SUB // 16):
                v = idx_v[l, k, pl.ds(16 * g, 16)]
                rem = v & (TC_COLS - 1)
                idx_v[l, k, pl.ds(16 * g, 16)] = (
                    (v - rem) + 4 * (rem & (TC_SUB - 1)) + (rem >> 13))
        return carry
    lax.fori_loop(0, HIST, rbody, 0)

    lane = lax.iota(jnp.int32, 16)
    rows_h = [lane + 16 * h for h in range(2)]

    def issue_gather(l, b):
        for k in range(N_SUB):
            pltpu.async_copy(w_hbm.at[idx_v.at[l, k]],
                             stagings[b].at[pl.ds(k * IDX_SUB, IDX_SUB)],
                             sem_g[b])

    def wait_gather(b):
        pltpu.make_async_copy(w_hbm.at[pl.ds(0, B_PER_W)],
                              stagings[b], sem_g[b]).wait()

    def out_copy(l, b):
        return pltpu.make_async_copy(
            outbufs[b].at[:, pl.ds(0, B_PER_W)],
            out_hbm.at[pl.ds(l * EMBED, EMBED), pl.ds(wid * B_PER_W, B_PER_W)],
            sem_o[b])

    for b in range(NBUF):
        issue_gather(b, b)

    def body(r, carry):
        for b in range(NBUF):
            l = r * NBUF + b

            @pl.when(l < HIST)
            def _step():
                wait_gather(b)

                @pl.when(r > 0)
                def _wait_prev_out():
                    out_copy(l - NBUF, b).wait()

                for bp in range(B_PER_W):
                    col = jnp.full((16,), bp, jnp.int32)
                    for h in range(2):
                        vals = stagings[b][bp, pl.ds(16 * h, 16)]
                        plsc.store_scatter(outbufs[b], [rows_h[h], col], vals)

                out_copy(l, b).start()

                @pl.when(l + NBUF < HIST)
                def _issue_next():
                    issue_gather(l + NBUF, b)
        return carry

    lax.fori_loop(0, N_ROUNDS, body, 0)

    for l in range(HIST - NBUF, HIST):
        out_copy(l, l % NBUF).wait()


@jax.jit
def kernel(x, W):
    x4 = x.astype(jnp.int32).T.reshape(HIST, NW, N_SUB, IDX_SUB)
    w_rm = _compact_table(W.T).reshape(N_PAD, EMBED)
    mesh = plsc.VectorSubcoreMesh(core_axis_name="c", subcore_axis_name="s")
    scratch = (
        [pltpu.VMEM((HIST, N_SUB, IDX_SUB), jnp.int32)]
        + [pltpu.VMEM((B_PER_W, EMBED), jnp.float32)] * NBUF
        + [pltpu.VMEM((EMBED, OB_PITCH), jnp.float32)] * NBUF
        + [pltpu.SemaphoreType.DMA] * (2 * NBUF)
    )
    run = pl.kernel(
        _sc_kernel,
        out_type=jax.ShapeDtypeStruct((HIST * EMBED, BATCH), jnp.float32),
        mesh=mesh,
        scratch_types=scratch,
        compiler_params=pltpu.CompilerParams(
            needs_layout_passes=False, use_tc_tiling_on_sc=False),
    )
    out = run(x4, w_rm)
    return out.reshape(HIST, EMBED, BATCH).transpose(2, 1, 0)
